# baseline (speedup 1.0000x reference)
import functools

import jax
import jax.numpy as jnp
from jax import lax
from jax.experimental import pallas as pl
from jax.experimental.pallas import tpu as pltpu
from jax.experimental.pallas import tpu_sc as plsc

IN_DIM = 3
M = 3
N = 32
C = 2
W = 4096
PAD_L = 2
TW = W + 8

BB = 8192
NG = 4
NL = N // NG
NR = 8
CR = 1024

_INV2PI = 0.15915494309189535
_MAGIC = 12582912.0
_C1 = 6.28125
_C2 = 0.0019353071795864769
_SIN_P = (
    9.9999999420e-01,
    -1.6666664500e-01,
    8.3333096487e-03,
    -1.9840126801e-04,
    2.7528926525e-06,
    -2.4672325863e-08,
    1.3435869084e-10,
)


def _fast_sin(a):
    n = a * _INV2PI + _MAGIC - _MAGIC
    r = a - n * _C1 - n * _C2
    r2 = r * r
    p = _SIN_P[6]
    for k in (5, 4, 3, 2, 1, 0):
        p = p * r2 + _SIN_P[k]
    return p * r


_NV = M * N // 8


def _coord_body(xr_ref, g_ref, h_ref, o_ref):
    gb = g_ref[...]
    hb = h_ref[...]

    @pl.loop(0, BB, step=128)
    def _tile(i):
        xs = xr_ref[:, :, pl.ds(i, 128)]
        a = hb + gb[0] * xs[0][None]
        a += gb[1] * xs[1][None]
        a += gb[2] * xs[2][None]
        s = _fast_sin(a)
        gx = s[0:4] * s[4:8] * s[8:12]
        ix = ((gx + 1.0) * W - 1.0) * 0.5
        jj = i >> 7
        for g in range(NG):
            o_ref[g, jj, :, :] = ix[g]


def _coords(xrep, gb, hb, bn):
    return pl.pallas_call(
        _coord_body,
        grid=(bn // BB,),
        in_specs=[
            pl.BlockSpec((IN_DIM, 8, BB), lambda i: (0, 0, i)),
            pl.BlockSpec((IN_DIM, _NV, 8, 128), lambda i: (0, 0, 0, 0)),
            pl.BlockSpec((_NV, 8, 128), lambda i: (0, 0, 0)),
        ],
        out_specs=pl.BlockSpec(
            (NG, BB // 128, NL, 128), lambda i: (0, i, 0, 0)
        ),
        out_shape=jax.ShapeDtypeStruct(
            (NG, bn // 128, NL, 128), jnp.float32
        ),
    )(xrep, gb, hb)


def _interp_call(ix_lin, tabs, bn):
    rt = bn // NR
    nch = rt // CR
    mesh = plsc.VectorSubcoreMesh(
        core_axis_name="c", subcore_axis_name="s", num_cores=2, num_subcores=16
    )

    @functools.partial(
        pl.kernel,
        out_type=jax.ShapeDtypeStruct((bn * N * C // 128, 128), jnp.float32),
        mesh=mesh,
        scratch_types=[
            pltpu.VMEM((C, NL, TW), jnp.float32),
            pltpu.VMEM((2, CR // 128, NL, 128), jnp.float32),
            pltpu.VMEM((2, 2, CR // 2, NL * C), jnp.float32),
            pltpu.SemaphoreType.DMA((2,)),
            pltpu.SemaphoreType.DMA((2,)),
        ],
        compiler_params=pltpu.CompilerParams(
            use_tc_tiling_on_sc=False, needs_layout_passes=False
        ),
    )
    def run(ix_hbm, tab_hbm, out_hbm, tab_v, ix_v, out_v, isem, osem):
        wid = lax.axis_index("s") * 2 + lax.axis_index("c")
        grp = wid % NG
        rng = wid // NG
        rows0 = rng * rt
        pltpu.sync_copy(tab_hbm.at[:, pl.ds(grp * NL, NL), :], tab_v)

        lane = lax.iota(jnp.int32, 16)
        lvl = lane & (NL - 1)
        coloff = lane >> 3
        ch0 = lvl * C
        czero = jnp.zeros((16,), jnp.int32)
        cone = czero + 1

        colt0 = rng * (rt // 128)

        def in_copy(j, s):
            return pltpu.make_async_copy(
                ix_hbm.at[grp, pl.ds(colt0 + j * (CR // 128), CR // 128), :, :],
                ix_v.at[s],
                isem.at[s],
            )

        def out_copy(j, s, p):
            return pltpu.make_async_copy(
                out_v.at[s, p],
                out_hbm.at[
                    pl.ds((rows0 + j * CR) // 2, CR // 2),
                    pl.ds(p * (N * C) + grp * NL * C, NL * C),
                ],
                osem.at[s],
            )

        in_copy(0, 0).start()
        in_copy(1, 1).start()

        @pl.loop(0, nch, step=2)
        def _chunk(j0):
            for s in (0, 1):
                j = j0 + s

                @pl.when(j0 >= 2)
                def _():
                    out_copy(j - 2, s, 0).wait()
                    out_copy(j - 2, s, 1).wait()

                in_copy(j, s).wait()

                @pl.loop(0, CR // 128, step=1)
                def _jt(jt):
                    jtv = czero + jt
                    jcol = jt * 128

                    @plsc.parallel_loop(0, 128, step=2, unroll=8)
                    def _vec(c):
                        colv = coloff + c
                        ix = plsc.load_gather(
                            ix_v.at[s], [jtv, lvl, colv]
                        )
                        ixp1 = ix + 1.0
                        base = ixp1.astype(jnp.int32)
                        t = ixp1 - base.astype(jnp.float32)
                        t2 = t * t
                        t3 = t2 * t
                        w0 = -0.75 * (t3 - 2.0 * t2 + t)
                        w3 = -0.75 * (t2 - t3)
                        w1 = 1.25 * t3 - 2.25 * t2 + 1.0
                        w2 = 1.0 - w0 - w1 - w3
                        v00 = plsc.load_gather(tab_v, [czero, lvl, base])
                        v01 = plsc.load_gather(tab_v, [cone, lvl, base])
                        v10 = plsc.load_gather(tab_v, [czero, lvl, base + 1])
                        v11 = plsc.load_gather(tab_v, [cone, lvl, base + 1])
                        v20 = plsc.load_gather(tab_v, [czero, lvl, base + 2])
                        v21 = plsc.load_gather(tab_v, [cone, lvl, base + 2])
                        v30 = plsc.load_gather(tab_v, [czero, lvl, base + 3])
                        v31 = plsc.load_gather(tab_v, [cone, lvl, base + 3])
                        acc0 = w0 * v00 + w1 * v10 + w2 * v20 + w3 * v30
                        acc1 = w0 * v01 + w1 * v11 + w2 * v21 + w3 * v31
                        rowh = czero + ((jcol + c) >> 1)
                        plsc.store_scatter(
                            out_v.at[s], [coloff, rowh, ch0], acc0
                        )
                        plsc.store_scatter(
                            out_v.at[s], [coloff, rowh, ch0 + 1], acc1
                        )

                out_copy(j, s, 0).start()
                out_copy(j, s, 1).start()

                @pl.when(j + 2 < nch)
                def _():
                    in_copy(j + 2, s).start()

        out_copy(nch - 2, 0, 0).wait()
        out_copy(nch - 2, 0, 1).wait()
        out_copy(nch - 1, 1, 0).wait()
        out_copy(nch - 1, 1, 1).wait()

    return run(ix_lin, tabs)


def kernel(x, grids, G, H, size):
    bn = x.shape[0]
    xt = x.T
    xrep = jnp.broadcast_to(xt[:, None, :], (IN_DIM, 8, bn))
    gb = jnp.broadcast_to(
        G.reshape(IN_DIM, M * N).reshape(IN_DIM, _NV, 8, 1),
        (IN_DIM, _NV, 8, 128),
    )
    hb = jnp.broadcast_to(H.reshape(M * N).reshape(_NV, 8, 1), (_NV, 8, 128))
    tabs = jnp.pad(
        jnp.transpose(grids, (1, 0, 2)), ((0, 0), (0, 0), (PAD_L, TW - W - PAD_L))
    )
    ix_lin = _coords(xrep, gb, hb, bn)
    out_lin = _interp_call(ix_lin, tabs, bn)
    return out_lin.reshape(bn, N * C)


# speedup vs baseline: 1.2612x; 1.2612x over previous
import functools

import jax
import jax.numpy as jnp
from jax import lax
from jax.experimental import pallas as pl
from jax.experimental.pallas import tpu as pltpu
from jax.experimental.pallas import tpu_sc as plsc

IN_DIM = 3
M = 3
N = 32
C = 2
W = 4096
PAD_L = 2
TW = W + 8

BB = 8192
NG = 4
NL = N // NG
NR = 8
CR = 1024

_INV2PI = 0.15915494309189535
_MAGIC = 12582912.0
_C1 = 6.28125
_C2 = 0.0019353071795864769
_SIN_P = (
    9.9999999420e-01,
    -1.6666664500e-01,
    8.3333096487e-03,
    -1.9840126801e-04,
    2.7528926525e-06,
    -2.4672325863e-08,
    1.3435869084e-10,
)


def _fast_sin(a):
    n = a * _INV2PI + _MAGIC - _MAGIC
    r = a - n * _C1 - n * _C2
    r2 = r * r
    p = _SIN_P[6]
    for k in (5, 4, 3, 2, 1, 0):
        p = p * r2 + _SIN_P[k]
    return p * r


_NV = M * N // 8


def _coord_body(xr_ref, g_ref, h_ref, o_ref):
    gb = g_ref[...]
    hb = h_ref[...]

    @pl.loop(0, BB, step=128)
    def _tile(i):
        xs = xr_ref[:, :, pl.ds(i, 128)]
        a = hb + gb[0] * xs[0][None]
        a += gb[1] * xs[1][None]
        a += gb[2] * xs[2][None]
        s = _fast_sin(a)
        gx = s[0:4] * s[4:8] * s[8:12]
        ix = ((gx + 1.0) * W - 1.0) * 0.5
        jj = i >> 7
        for g in range(NG):
            o_ref[g, jj, :, :] = ix[g]


def _coords(xrep, gb, hb, bn):
    return pl.pallas_call(
        _coord_body,
        grid=(bn // BB,),
        in_specs=[
            pl.BlockSpec((IN_DIM, 8, BB), lambda i: (0, 0, i)),
            pl.BlockSpec((IN_DIM, _NV, 8, 128), lambda i: (0, 0, 0, 0)),
            pl.BlockSpec((_NV, 8, 128), lambda i: (0, 0, 0)),
        ],
        out_specs=pl.BlockSpec(
            (NG, BB // 128, NL, 128), lambda i: (0, i, 0, 0)
        ),
        out_shape=jax.ShapeDtypeStruct(
            (NG, bn // 128, NL, 128), jnp.float32
        ),
    )(xrep, gb, hb)


def _interp_call(ix_lin, tabs, bn):
    rt = bn // NR
    nch = rt // CR
    mesh = plsc.VectorSubcoreMesh(
        core_axis_name="c", subcore_axis_name="s", num_cores=2, num_subcores=16
    )

    @functools.partial(
        pl.kernel,
        out_type=jax.ShapeDtypeStruct((bn * N * C // 128, 128), jnp.float32),
        mesh=mesh,
        scratch_types=[
            pltpu.VMEM((NL, TW), jnp.int32),
            pltpu.VMEM((2, CR // 128, NL, 128), jnp.float32),
            pltpu.VMEM((2, 2, CR // 2, NL * C), jnp.float32),
            pltpu.SemaphoreType.DMA((2,)),
            pltpu.SemaphoreType.DMA((2,)),
        ],
        compiler_params=pltpu.CompilerParams(
            use_tc_tiling_on_sc=False, needs_layout_passes=False
        ),
    )
    def run(ix_hbm, tab_hbm, out_hbm, tab_v, ix_v, out_v, isem, osem):
        wid = lax.axis_index("s") * 2 + lax.axis_index("c")
        grp = wid % NG
        rng = wid // NG
        rows0 = rng * rt
        pltpu.sync_copy(tab_hbm.at[pl.ds(grp * NL, NL), :], tab_v)

        lane = lax.iota(jnp.int32, 16)
        lvl = lane & (NL - 1)
        coloff = lane >> 3
        ch0 = lvl * C
        czero = jnp.zeros((16,), jnp.int32)
        himask = czero - 65536

        colt0 = rng * (rt // 128)

        def in_copy(j, s):
            return pltpu.make_async_copy(
                ix_hbm.at[grp, pl.ds(colt0 + j * (CR // 128), CR // 128), :, :],
                ix_v.at[s],
                isem.at[s],
            )

        def out_copy(j, s, p):
            return pltpu.make_async_copy(
                out_v.at[s, p],
                out_hbm.at[
                    pl.ds((rows0 + j * CR) // 2, CR // 2),
                    pl.ds(p * (N * C) + grp * NL * C, NL * C),
                ],
                osem.at[s],
            )

        in_copy(0, 0).start()
        in_copy(1, 1).start()

        @pl.loop(0, nch, step=2)
        def _chunk(j0):
            for s in (0, 1):
                j = j0 + s

                @pl.when(j0 >= 2)
                def _():
                    out_copy(j - 2, s, 0).wait()
                    out_copy(j - 2, s, 1).wait()

                in_copy(j, s).wait()

                @pl.loop(0, CR // 128, step=1)
                def _jt(jt):
                    jtv = czero + jt
                    jcol = jt * 128

                    @plsc.parallel_loop(0, 128, step=2, unroll=4)
                    def _vec(c):
                        colv = coloff + c
                        ix = plsc.load_gather(
                            ix_v.at[s], [jtv, lvl, colv]
                        )
                        ixp1 = ix + 1.0
                        base = ixp1.astype(jnp.int32)
                        t = ixp1 - base.astype(jnp.float32)
                        t2 = t * t
                        t3 = t2 * t
                        w0 = -0.75 * (t3 - 2.0 * t2 + t)
                        w3 = -0.75 * (t2 - t3)
                        w1 = 1.25 * t3 - 2.25 * t2 + 1.0
                        w2 = 1.0 - w0 - w1 - w3
                        p0 = plsc.load_gather(tab_v, [lvl, base])
                        p1 = plsc.load_gather(tab_v, [lvl, base + 1])
                        p2 = plsc.load_gather(tab_v, [lvl, base + 2])
                        p3 = plsc.load_gather(tab_v, [lvl, base + 3])
                        v00 = plsc.bitcast(p0 << 16, jnp.float32)
                        v01 = plsc.bitcast(p0 & himask, jnp.float32)
                        v10 = plsc.bitcast(p1 << 16, jnp.float32)
                        v11 = plsc.bitcast(p1 & himask, jnp.float32)
                        v20 = plsc.bitcast(p2 << 16, jnp.float32)
                        v21 = plsc.bitcast(p2 & himask, jnp.float32)
                        v30 = plsc.bitcast(p3 << 16, jnp.float32)
                        v31 = plsc.bitcast(p3 & himask, jnp.float32)
                        acc0 = w0 * v00 + w1 * v10 + w2 * v20 + w3 * v30
                        acc1 = w0 * v01 + w1 * v11 + w2 * v21 + w3 * v31
                        rowh = czero + ((jcol + c) >> 1)
                        plsc.store_scatter(
                            out_v.at[s], [coloff, rowh, ch0], acc0
                        )
                        plsc.store_scatter(
                            out_v.at[s], [coloff, rowh, ch0 + 1], acc1
                        )

                out_copy(j, s, 0).start()
                out_copy(j, s, 1).start()

                @pl.when(j + 2 < nch)
                def _():
                    in_copy(j + 2, s).start()

        out_copy(nch - 2, 0, 0).wait()
        out_copy(nch - 2, 0, 1).wait()
        out_copy(nch - 1, 1, 0).wait()
        out_copy(nch - 1, 1, 1).wait()

    return run(ix_lin, tabs)


def kernel(x, grids, G, H, size):
    bn = x.shape[0]
    xt = x.T
    xrep = jnp.broadcast_to(xt[:, None, :], (IN_DIM, 8, bn))
    gb = jnp.broadcast_to(
        G.reshape(IN_DIM, M * N).reshape(IN_DIM, _NV, 8, 1),
        (IN_DIM, _NV, 8, 128),
    )
    hb = jnp.broadcast_to(H.reshape(M * N).reshape(_NV, 8, 1), (_NV, 8, 128))
    gbits = jax.lax.bitcast_convert_type(
        grids.astype(jnp.bfloat16), jnp.uint16
    ).astype(jnp.uint32)
    packed = jax.lax.bitcast_convert_type(
        gbits[:, 0, :] | (gbits[:, 1, :] << 16), jnp.int32
    )
    tabs = jnp.pad(packed, ((0, 0), (PAD_L, TW - W - PAD_L)))
    ix_lin = _coords(xrep, gb, hb, bn)
    out_lin = _interp_call(ix_lin, tabs, bn)
    return out_lin.reshape(bn, N * C)


# speedup vs baseline: 1.3032x; 1.0333x over previous
import functools

import jax
import jax.numpy as jnp
from jax import lax
from jax.experimental import pallas as pl
from jax.experimental.pallas import tpu as pltpu
from jax.experimental.pallas import tpu_sc as plsc

IN_DIM = 3
M = 3
N = 32
C = 2
W = 4096
PAD_L = 2
TW = W + 8

BB = 8192
NG = 4
NL = N // NG
NR = 8
CR = 1024

_INV2PI = 0.15915494309189535
_MAGIC = 12582912.0
_C1 = 6.28125
_C2 = 0.0019353071795864769
_SIN_P = (
    9.9999999420e-01,
    -1.6666664500e-01,
    8.3333096487e-03,
    -1.9840126801e-04,
    2.7528926525e-06,
    -2.4672325863e-08,
    1.3435869084e-10,
)


def _fast_sin(a):
    n = a * _INV2PI + _MAGIC - _MAGIC
    r = a - n * _C1 - n * _C2
    r2 = r * r
    p = _SIN_P[6]
    for k in (5, 4, 3, 2, 1, 0):
        p = p * r2 + _SIN_P[k]
    return p * r


_NV = M * N // 8


def _coord_body(xr_ref, g_ref, h_ref, o_ref):
    gb = g_ref[...]
    hb = h_ref[...]

    @pl.loop(0, BB, step=128)
    def _tile(i):
        xs = xr_ref[:, :, pl.ds(i, 128)]
        a = hb + gb[0] * xs[0][None]
        a += gb[1] * xs[1][None]
        a += gb[2] * xs[2][None]
        s = _fast_sin(a)
        gx = s[0:4] * s[4:8] * s[8:12]
        ix = ((gx + 1.0) * W - 1.0) * 0.5
        jj = i >> 7
        for g in range(NG):
            o_ref[g, jj, :, :] = ix[g]


def _coords(xrep, gb, hb, bn):
    return pl.pallas_call(
        _coord_body,
        grid=(bn // BB,),
        in_specs=[
            pl.BlockSpec((IN_DIM, 8, BB), lambda i: (0, 0, i)),
            pl.BlockSpec((IN_DIM, _NV, 8, 128), lambda i: (0, 0, 0, 0)),
            pl.BlockSpec((_NV, 8, 128), lambda i: (0, 0, 0)),
        ],
        out_specs=pl.BlockSpec(
            (NG, BB // 128, NL, 128), lambda i: (0, i, 0, 0)
        ),
        out_shape=jax.ShapeDtypeStruct(
            (NG, bn // 128, NL, 128), jnp.float32
        ),
    )(xrep, gb, hb)


def _interp_call(ix_lin, tabs, bn):
    rt = bn // NR
    nch = rt // CR
    mesh = plsc.VectorSubcoreMesh(
        core_axis_name="c", subcore_axis_name="s", num_cores=2, num_subcores=16
    )

    @functools.partial(
        pl.kernel,
        out_type=jax.ShapeDtypeStruct((bn * N * C // 128, 128), jnp.float32),
        mesh=mesh,
        scratch_types=[
            pltpu.VMEM((C, NL, TW), jnp.float32),
            pltpu.VMEM((2, CR // 128, NL, 128), jnp.float32),
            pltpu.VMEM((2, 2, CR // 2, NL * C), jnp.float32),
            pltpu.SemaphoreType.DMA((2,)),
            pltpu.SemaphoreType.DMA((2,)),
        ],
        compiler_params=pltpu.CompilerParams(
            use_tc_tiling_on_sc=False, needs_layout_passes=False
        ),
    )
    def run(ix_hbm, tab_hbm, out_hbm, tab_v, ix_v, out_v, isem, osem):
        wid = lax.axis_index("s") * 2 + lax.axis_index("c")
        grp = wid % NG
        rng = wid // NG
        rows0 = rng * rt
        pltpu.sync_copy(tab_hbm.at[:, pl.ds(grp * NL, NL), :], tab_v)

        lane = lax.iota(jnp.int32, 16)
        lvl = lane & (NL - 1)
        coloff = lane >> 3
        ch0 = lvl * C
        czero = jnp.zeros((16,), jnp.int32)
        cone = czero + 1

        colt0 = rng * (rt // 128)

        def in_copy(j, s):
            return pltpu.make_async_copy(
                ix_hbm.at[grp, pl.ds(colt0 + j * (CR // 128), CR // 128), :, :],
                ix_v.at[s],
                isem.at[s],
            )

        def out_copy(j, s, p):
            return pltpu.make_async_copy(
                out_v.at[s, p],
                out_hbm.at[
                    pl.ds((rows0 + j * CR) // 2, CR // 2),
                    pl.ds(p * (N * C) + grp * NL * C, NL * C),
                ],
                osem.at[s],
            )

        in_copy(0, 0).start()
        in_copy(1, 1).start()

        @pl.loop(0, nch, step=2)
        def _chunk(j0):
            for s in (0, 1):
                j = j0 + s

                @pl.when(j0 >= 2)
                def _():
                    out_copy(j - 2, s, 0).wait()
                    out_copy(j - 2, s, 1).wait()

                in_copy(j, s).wait()

                @pl.loop(0, CR // 128, step=1)
                def _jt(jt):
                    jtv = czero + jt
                    jcol = jt * 128

                    @plsc.parallel_loop(0, 128, step=2, unroll=4)
                    def _vec(c):
                        colv = coloff + c
                        ix = plsc.load_gather(
                            ix_v.at[s], [jtv, lvl, colv]
                        )
                        ixp1 = ix + 1.0
                        base = ixp1.astype(jnp.int32)
                        t = ixp1 - base.astype(jnp.float32)
                        t2 = t * t
                        t3 = t2 * t
                        w0 = -0.75 * (t3 - 2.0 * t2 + t)
                        w3 = -0.75 * (t2 - t3)
                        w1 = 1.25 * t3 - 2.25 * t2 + 1.0
                        w2 = 1.0 - w0 - w1 - w3
                        v00 = plsc.load_gather(tab_v, [czero, lvl, base])
                        v01 = plsc.load_gather(tab_v, [cone, lvl, base])
                        v10 = plsc.load_gather(tab_v, [czero, lvl, base + 1])
                        v11 = plsc.load_gather(tab_v, [cone, lvl, base + 1])
                        v20 = plsc.load_gather(tab_v, [czero, lvl, base + 2])
                        v21 = plsc.load_gather(tab_v, [cone, lvl, base + 2])
                        v30 = plsc.load_gather(tab_v, [czero, lvl, base + 3])
                        v31 = plsc.load_gather(tab_v, [cone, lvl, base + 3])
                        acc0 = w0 * v00 + w1 * v10 + w2 * v20 + w3 * v30
                        acc1 = w0 * v01 + w1 * v11 + w2 * v21 + w3 * v31
                        rowh = czero + ((jcol + c) >> 1)
                        plsc.store_scatter(
                            out_v.at[s], [coloff, rowh, ch0], acc0
                        )
                        plsc.store_scatter(
                            out_v.at[s], [coloff, rowh, ch0 + 1], acc1
                        )

                out_copy(j, s, 0).start()
                out_copy(j, s, 1).start()

                @pl.when(j + 2 < nch)
                def _():
                    in_copy(j + 2, s).start()

        out_copy(nch - 2, 0, 0).wait()
        out_copy(nch - 2, 0, 1).wait()
        out_copy(nch - 1, 1, 0).wait()
        out_copy(nch - 1, 1, 1).wait()

    return run(ix_lin, tabs)


def kernel(x, grids, G, H, size):
    bn = x.shape[0]
    xt = x.T
    xrep = jnp.broadcast_to(xt[:, None, :], (IN_DIM, 8, bn))
    gb = jnp.broadcast_to(
        G.reshape(IN_DIM, M * N).reshape(IN_DIM, _NV, 8, 1),
        (IN_DIM, _NV, 8, 128),
    )
    hb = jnp.broadcast_to(H.reshape(M * N).reshape(_NV, 8, 1), (_NV, 8, 128))
    tabs = jnp.pad(
        jnp.transpose(grids, (1, 0, 2)), ((0, 0), (0, 0), (PAD_L, TW - W - PAD_L))
    )
    ix_lin = _coords(xrep, gb, hb, bn)
    out_lin = _interp_call(ix_lin, tabs, bn)
    return out_lin.reshape(bn, N * C)
